# baseline (speedup 1.0000x reference)
import jax
import jax.numpy as jnp
from jax import lax
from jax.experimental import pallas as pl
from jax.experimental.pallas import tpu as pltpu
from jax.experimental.pallas import tpu_sc as plsc

N = 50000
D = 256
G = 512
NC = 2
NS = 16
NW = NC * NS
SPAN = 1568
C = 128
MAXPAIR = 7
NJ = D // 16


def _sc_body(feats, seg, w_in, b_in, zeros_in, part_out, aw_out,
             rb0, rb1, awb, wv, bv, accb, fbuf, fidx,
             sb0, sb1, curs,
             acc_sh, sr0, sr1, ss0, ss1):
    cid = lax.axis_index("c")
    sid = lax.axis_index("s")
    wid = sid * NC + cid

    pltpu.sync_copy(w_in, wv)
    pltpu.sync_copy(b_in, bv)

    pltpu.sync_copy(zeros_in, acc_sh.at[pl.ds(sid * (G // NS), G // NS)])
    plsc.subcore_barrier()

    start = wid * SPAN
    end = jnp.minimum(start + SPAN, N)
    nrows = end - start
    nfull = nrows // C
    rem = nrows - nfull * C
    nch = jnp.where(rem > 0, nfull + 1, nfull)
    masklo = start + nfull * C

    def row0_of(cc):
        return jnp.where(cc < nfull, start + cc * C, end - C)

    rbufs = (rb0, rb1)
    sbufs = (sb0, sb1)
    rsems = (sr0, sr1)
    ssems = (ss0, ss1)

    def start_dma(cc, b):
        r0 = row0_of(cc)
        pltpu.make_async_copy(feats.at[pl.ds(r0, C)], rbufs[b], rsems[b]).start()
        pltpu.make_async_copy(seg.at[pl.ds(r0, C)], sbufs[b], ssems[b]).start()

    start_dma(0, 0)
    start_dma(1, 1)

    iota16 = lax.broadcasted_iota(jnp.int32, (16,), 0)
    breg = bv[...]
    lane0 = iota16 == 0
    lane15 = iota16 == 15
    izero = jnp.zeros((16,), jnp.int32)
    zero = jnp.zeros((16,), jnp.float32)

    curs[0] = -1

    def zacc(j, _):
        accb[pl.ds(j * 16, 16)] = zero
        return 0
    lax.fori_loop(0, NJ, zacc, 0)

    def do_chunk(cc, b):
        r0 = row0_of(cc)
        rb = rbufs[b]
        sb = sbufs[b]
        pltpu.make_async_copy(feats.at[pl.ds(r0, C)], rb, rsems[b]).wait()
        pltpu.make_async_copy(seg.at[pl.ds(r0, C)], sb, ssems[b]).wait()
        off = r0 - start
        wregs = [wv[pl.ds(j * 16, 16)] for j in range(NJ)]

        carry_in = tuple(accb[pl.ds(j * 16, 16)] for j in range(NJ)) + (curs[0],)

        def group_body(g, carry):
            accs = list(carry[:NJ])
            cur = carry[NJ]
            svg = sb[pl.ds(g * 16, 16)]
            for r in range(16):
                rr = g * 16 + r
                a = [zero, zero, zero, zero]
                for j in range(NJ):
                    v = rb[rr, pl.ds(j * 16, 16)]
                    a[j % 4] = a[j % 4] + v * wregs[j]
                tot = (a[0] + a[1]) + (a[2] + a[3])
                cs = plsc.cumsum(tot)
                awv = cs + breg
                plsc.store_scatter(awb, [jnp.full((16,), off + rr, jnp.int32)],
                                   awv, mask=lane15)
                sigv = 1.0 / (1.0 + jnp.exp(-awv))
                ws = sigv[15]
                keep = (cc < nfull) | ((r0 + rr) >= masklo)
                ws = jnp.where(keep, ws, 0.0)

                seg_s = svg[r]
                do_flush = (seg_s != cur) & (cur >= 0)
                flushed = accs

                @pl.when(do_flush)
                def _(flushed=flushed, cur=cur):
                    for j in range(NJ):
                        fbuf[0, pl.ds(j * 16, 16)] = flushed[j]
                    plsc.store_scatter(fidx, [izero],
                                       jnp.full((16,), cur, jnp.int32),
                                       mask=lane0)
                    pltpu.sync_copy(fbuf, acc_sh.at[fidx], add=True)

                accs = [jnp.where(do_flush, 0.0, acc) for acc in accs]
                cur = jnp.where(do_flush | (cur < 0), seg_s, cur)
                for j in range(NJ):
                    v = rb[rr, pl.ds(j * 16, 16)]
                    accs[j] = accs[j] + v * ws
            return tuple(accs) + (cur,)

        carry_out = lax.fori_loop(0, C // 16, group_body, carry_in)
        for j in range(NJ):
            accb[pl.ds(j * 16, 16)] = carry_out[j]
        curs[0] = carry_out[NJ]

    def pair(i, _):
        for b in range(2):
            cc = i * 2 + b

            @pl.when(cc < nch)
            def _():
                do_chunk(cc, b)

                @pl.when(cc + 2 < nch)
                def _():
                    start_dma(cc + 2, b)
        return 0
    lax.fori_loop(0, MAXPAIR, pair, 0)

    @pl.when(curs[0] >= 0)
    def _():
        def cp(j, _):
            fbuf[0, pl.ds(j * 16, 16)] = accb[pl.ds(j * 16, 16)]
            return 0
        lax.fori_loop(0, NJ, cp, 0)
        plsc.store_scatter(fidx, [izero],
                           jnp.full((16,), curs[0], jnp.int32), mask=lane0)
        pltpu.sync_copy(fbuf, acc_sh.at[fidx], add=True)

    pltpu.sync_copy(awb, aw_out.at[pl.ds(start, SPAN)])

    plsc.subcore_barrier()
    rows_per = G // NS
    pltpu.sync_copy(acc_sh.at[pl.ds(sid * rows_per, rows_per)],
                    part_out.at[pl.ds(cid * G + sid * rows_per, rows_per)])


def _combine_body(p_ref, o_ref):
    o_ref[...] = p_ref[0:G, :] + p_ref[G:2 * G, :]


def kernel(feats, segment_ids, W, b):
    w_flat = W.reshape(D).astype(jnp.float32)
    b_pad = jnp.broadcast_to(b.astype(jnp.float32), (16,))
    seg = segment_ids.astype(jnp.int32)
    zeros_in = jnp.zeros((G // NS, D), jnp.float32)
    mesh = plsc.VectorSubcoreMesh(core_axis_name="c", subcore_axis_name="s",
                                  num_cores=NC, num_subcores=NS)
    sc = pl.kernel(
        _sc_body,
        out_type=(jax.ShapeDtypeStruct((NC * G, D), jnp.float32),
                  jax.ShapeDtypeStruct((NW * SPAN,), jnp.float32)),
        mesh=mesh,
        compiler_params=pltpu.CompilerParams(use_tc_tiling_on_sc=False,
                                             needs_layout_passes=False),
        scratch_types=[
            pltpu.VMEM((C, D), jnp.float32),
            pltpu.VMEM((C, D), jnp.float32),
            pltpu.VMEM((SPAN,), jnp.float32),
            pltpu.VMEM((D,), jnp.float32),
            pltpu.VMEM((16,), jnp.float32),
            pltpu.VMEM((D,), jnp.float32),
            pltpu.VMEM((1, D), jnp.float32),
            pltpu.VMEM((1,), jnp.int32),
            pltpu.VMEM((C,), jnp.int32),
            pltpu.VMEM((C,), jnp.int32),
            pltpu.SMEM((1,), jnp.int32),
            pltpu.VMEM_SHARED((G, D), jnp.float32),
            pltpu.SemaphoreType.DMA,
            pltpu.SemaphoreType.DMA,
            pltpu.SemaphoreType.DMA,
            pltpu.SemaphoreType.DMA,
        ],
    )
    part, aw_pad = sc(feats, seg, w_flat, b_pad, zeros_in)
    h = pl.pallas_call(
        _combine_body,
        out_shape=jax.ShapeDtypeStruct((G, D), jnp.float32),
    )(part)
    aw = aw_pad[:N].reshape(N, 1)
    return (h, aw)


# speedup vs baseline: 1.4851x; 1.4851x over previous
import jax
import jax.numpy as jnp
from jax import lax
from jax.experimental import pallas as pl
from jax.experimental.pallas import tpu as pltpu
from jax.experimental.pallas import tpu_sc as plsc

N = 50000
D = 256
G = 512
NC = 2
NS = 16
NW = NC * NS
SPAN = 1568
C = 128
GROUPS = C // 16
MAXPAIR = 7


def _sc_body(feats, seg, w_in, b_in, zeros_in, part_out, aw_out,
             rb0, rb1, rb2, sb0, sb1, sb2, awb, wv, bv,
             acc_sh, sr0, sr1, sr2, ss0, ss1, ss2, cs0, cs1, cs2):
    cid = lax.axis_index("c")
    sid = lax.axis_index("s")
    wid = sid * NC + cid

    pltpu.sync_copy(w_in, wv)
    pltpu.sync_copy(b_in, bv)

    pltpu.sync_copy(zeros_in, acc_sh.at[pl.ds(sid * (G // NS), G // NS)])
    plsc.subcore_barrier()

    start = wid * SPAN
    end = jnp.minimum(start + SPAN, N)
    nrows = end - start
    nfull = nrows // C
    rem = nrows - nfull * C
    nch = jnp.where(rem > 0, nfull + 1, nfull)
    masklo = start + nfull * C

    def row0_of(cc):
        return jnp.where(cc < nfull, start + cc * C, end - C)

    rbufs = (rb0, rb1, rb2)
    sbufs = (sb0, sb1, sb2)
    rsems = (sr0, sr1, sr2)
    ssems = (ss0, ss1, ss2)
    csems = (cs0, cs1, cs2)

    def start_dma(cc, b):
        r0 = row0_of(cc)
        pltpu.make_async_copy(feats.at[pl.ds(r0, C)], rbufs[b], rsems[b]).start()
        pltpu.make_async_copy(seg.at[pl.ds(r0, C)], sbufs[b], ssems[b]).start()

    def drain_scatter(b):
        pltpu.make_async_copy(rbufs[b], acc_sh.at[sbufs[b]], csems[b]).wait()

    start_dma(0, 0)
    start_dma(1, 1)

    iota16 = lax.broadcasted_iota(jnp.int32, (16,), 0)
    breg = bv[...]

    def do_chunk(cc, b):
        r0 = row0_of(cc)
        rb = rbufs[b]
        pltpu.make_async_copy(feats.at[pl.ds(r0, C)], rb, rsems[b]).wait()
        pltpu.make_async_copy(seg.at[pl.ds(r0, C)], sbufs[b], ssems[b]).wait()
        off = r0 - start
        zero = jnp.zeros((16,), jnp.float32)

        def group_body(g, _):
            rows16 = iota16 + g * 16

            kinit = tuple((iota16 + j) & (D - 1) for j in range(4))

            def dot_body(t, carry):
                a0, a1, a2, a3, k0, k1, k2, k3 = carry
                c0 = plsc.load_gather(rb, [rows16, k0])
                w0 = plsc.load_gather(wv, [k0])
                c1 = plsc.load_gather(rb, [rows16, k1])
                w1 = plsc.load_gather(wv, [k1])
                c2 = plsc.load_gather(rb, [rows16, k2])
                w2 = plsc.load_gather(wv, [k2])
                c3 = plsc.load_gather(rb, [rows16, k3])
                w3 = plsc.load_gather(wv, [k3])
                return (a0 + c0 * w0, a1 + c1 * w1, a2 + c2 * w2, a3 + c3 * w3,
                        (k0 + 4) & (D - 1), (k1 + 4) & (D - 1),
                        (k2 + 4) & (D - 1), (k3 + 4) & (D - 1))
            a0, a1, a2, a3, *_ = lax.fori_loop(
                0, D // 4, dot_body, (zero, zero, zero, zero) + kinit,
                unroll=2)
            wacc = (a0 + a1) + (a2 + a3)
            aw = wacc + breg
            awb[pl.ds(off + g * 16, 16)] = aw
            keep = (jnp.full((16,), cc, jnp.int32) < nfull) | ((r0 + rows16) >= masklo)
            wsig = 1.0 / (1.0 + jnp.exp(-aw))
            wsig = jnp.where(keep, wsig, 0.0)

            for r in range(16):
                row = g * 16 + r
                wr = wsig[r]

                def scale_body(j, _, row=row, wr=wr):
                    sl = pl.ds(j * 16, 16)
                    rb[row, sl] = rb[row, sl] * wr
                    return 0
                lax.fori_loop(0, D // 16, scale_body, 0, unroll=8)
            return 0
        lax.fori_loop(0, GROUPS, group_body, 0)
        pltpu.async_copy(rb, acc_sh.at[sbufs[b]], csems[b], add=True)

    def tri(i, _):
        for b in range(3):
            cc = i * 3 + b

            @pl.when(cc < nch)
            def _():
                do_chunk(cc, b)

                @pl.when(cc >= 1)
                def _():
                    drain_scatter((b + 2) % 3)

                @pl.when(cc + 2 < nch)
                def _():
                    start_dma(cc + 2, (b + 2) % 3)
        return 0
    lax.fori_loop(0, 5, tri, 0)

    lastb = (nch - 1) % 3
    for b in range(3):
        @pl.when(lastb == b)
        def _(b=b):
            drain_scatter(b)

    pltpu.sync_copy(awb, aw_out.at[pl.ds(start, SPAN)])

    plsc.subcore_barrier()
    rows_per = G // NS
    pltpu.sync_copy(acc_sh.at[pl.ds(sid * rows_per, rows_per)],
                    part_out.at[pl.ds(cid * G + sid * rows_per, rows_per)])


def _combine_body(p_ref, o_ref):
    o_ref[...] = p_ref[0:G, :] + p_ref[G:2 * G, :]


def kernel(feats, segment_ids, W, b):
    w_flat = W.reshape(D).astype(jnp.float32)
    b_pad = jnp.broadcast_to(b.astype(jnp.float32), (16,))
    seg = segment_ids.astype(jnp.int32)
    zeros_in = jnp.zeros((G // NS, D), jnp.float32)
    mesh = plsc.VectorSubcoreMesh(core_axis_name="c", subcore_axis_name="s",
                                  num_cores=NC, num_subcores=NS)
    sc = pl.kernel(
        _sc_body,
        out_type=(jax.ShapeDtypeStruct((NC * G, D), jnp.float32),
                  jax.ShapeDtypeStruct((NW * SPAN,), jnp.float32)),
        mesh=mesh,
        compiler_params=pltpu.CompilerParams(use_tc_tiling_on_sc=False,
                                             needs_layout_passes=False),
        scratch_types=[
            pltpu.VMEM((C, D), jnp.float32),
            pltpu.VMEM((C, D), jnp.float32),
            pltpu.VMEM((C, D), jnp.float32),
            pltpu.VMEM((C,), jnp.int32),
            pltpu.VMEM((C,), jnp.int32),
            pltpu.VMEM((C,), jnp.int32),
            pltpu.VMEM((SPAN,), jnp.float32),
            pltpu.VMEM((D,), jnp.float32),
            pltpu.VMEM((16,), jnp.float32),
            pltpu.VMEM_SHARED((G, D), jnp.float32),
        ] + [pltpu.SemaphoreType.DMA] * 9,
    )
    part, aw_pad = sc(feats, seg, w_flat, b_pad, zeros_in)
    h = pl.pallas_call(
        _combine_body,
        out_shape=jax.ShapeDtypeStruct((G, D), jnp.float32),
    )(part)
    aw = aw_pad[:N].reshape(N, 1)
    return (h, aw)
